# baseline (speedup 1.0000x reference)
import functools

import jax
import jax.numpy as jnp
import numpy as np
from jax import lax
from jax.experimental import pallas as pl
from jax.experimental.pallas import tpu as pltpu
from jax.experimental.pallas import tpu_sc as plsc

_ROWS = 4096
_COLS_P = 1024
_BLK_C = 128
_N_WORKERS = 32
_CLS_PER_W = _COLS_P // _N_WORKERS
_CHUNK = 8
_UNROLL = 8


def _bce(x, z):
    return jnp.maximum(x, 0.0) - x * z + jnp.log1p(jnp.exp(-jnp.abs(x)))


def _order_key(x):
    u = lax.bitcast_convert_type(x, jnp.uint32)
    neg = (u >> 31).astype(jnp.bool_)
    return jnp.where(neg, ~u, u | jnp.uint32(0x80000000))


def _order_key_inv(t):
    u = jnp.where(t >= jnp.uint32(0x80000000), t ^ jnp.uint32(0x80000000), ~t)
    return lax.bitcast_convert_type(u, jnp.float32)


def _keys_and_masks(pred, tgt):
    pos = jnp.sum(tgt, axis=0, keepdims=True)
    pos_gt = (pos >= (_ROWS / 2)).astype(pred.dtype)
    majority = tgt == pos_gt
    g = jnp.where(pos_gt > 0.5, -pred, pred)
    ukey = jnp.where(majority, _order_key(g), jnp.uint32(0))
    return pos_gt, majority, ukey


def _stage1_kernel(pred_ref, tgt_ref, keys_ref, keys16_ref, kmin_ref,
                   posgt_ref):
    pred = pred_ref[...]
    tgt = tgt_ref[...]
    pos_gt, majority, ukey = _keys_and_masks(pred, tgt)
    kmin = jnp.sum(jnp.where(majority, 0, 1), axis=0, keepdims=True)
    ukey_t = ukey.T
    keys_ref[...] = ukey_t
    keys16_ref[...] = (ukey_t >> 16).astype(jnp.uint16)
    kmin_ref[...] = kmin.reshape(1, 1, _BLK_C)
    posgt_ref[...] = pos_gt.reshape(1, 1, _BLK_C)


def _stage1(pred_p, tgt_p):
    return pl.pallas_call(
        _stage1_kernel,
        grid=(_COLS_P // _BLK_C,),
        in_specs=[
            pl.BlockSpec((_ROWS, _BLK_C), lambda i: (0, i)),
            pl.BlockSpec((_ROWS, _BLK_C), lambda i: (0, i)),
        ],
        out_specs=[
            pl.BlockSpec((_BLK_C, _ROWS), lambda i: (i, 0)),
            pl.BlockSpec((_BLK_C, _ROWS), lambda i: (i, 0)),
            pl.BlockSpec((1, 1, _BLK_C), lambda i: (i, 0, 0)),
            pl.BlockSpec((1, 1, _BLK_C), lambda i: (i, 0, 0)),
        ],
        out_shape=[
            jax.ShapeDtypeStruct((_COLS_P, _ROWS), jnp.uint32),
            jax.ShapeDtypeStruct((_COLS_P, _ROWS), jnp.uint16),
            jax.ShapeDtypeStruct((_COLS_P // _BLK_C, 1, _BLK_C), jnp.int32),
            jax.ShapeDtypeStruct((_COLS_P // _BLK_C, 1, _BLK_C), jnp.float32),
        ],
    )(pred_p, tgt_p)


def _count16(kbuf16, cl, mid):
    midv = jnp.full((16,), mid)
    lomask = jnp.uint32(0xFFFF)
    sh16 = jnp.uint32(16)

    def inner(j, acc, cl=cl, midv=midv):
        b = j * (16 * _UNROLL)
        for u in range(_UNROLL):
            ks = kbuf16[cl, pl.ds(b + u * 16, 16)]
            acc = acc + plsc.all_reduce_population_count(
                (ks & lomask) >= midv
            )
            acc = acc + plsc.all_reduce_population_count(
                (ks >> sh16) >= midv
            )
        return acc

    acc = lax.fori_loop(
        0, (_ROWS // 2) // (16 * _UNROLL), inner, jnp.zeros((16,), jnp.int32)
    )
    return acc[0]


def _sc_search_class(kbuf, kbuf16, candbuf, cl, kk):
    one = jnp.uint32(1)

    def l1(i, st, cl=cl):
        lo, hi, cg = st
        last = i >= 16
        mid = lax.select(last, lo + one, lo + ((hi - lo + one) >> one))
        c = _count16(kbuf16, cl, mid)
        ok = jnp.logical_and(c >= kk, jnp.logical_not(last))
        return (
            lax.select(ok, mid, lo),
            lax.select(ok, hi, mid - one),
            lax.select(last, c, cg),
        )

    t16, _, cnt_gt16 = lax.fori_loop(
        0, 17, l1, (jnp.uint32(0), jnp.uint32(0xFF80), jnp.int32(0))
    )
    k2 = kk - cnt_gt16

    t16v = jnp.full((16,), t16, jnp.uint32)

    def comp(j, off, cl=cl, t16v=t16v):
        b = j * (16 * 2)
        for u in range(2):
            ks = kbuf[cl, pl.ds(b + u * 16, 16)]
            m = (ks >> jnp.uint32(16)) == t16v
            plsc.store_compressed(candbuf.at[pl.ds(off, 16)], ks, mask=m)
            off = off + plsc.all_reduce_population_count(m)[0]
        return off

    nb = lax.fori_loop(0, _ROWS // (16 * 2), comp, jnp.int32(0))
    candbuf[pl.ds(nb, 16)] = jnp.zeros((16,), jnp.uint32)
    nvr = (nb + 15) >> 4

    def l2(_, lohi):
        lo, hi = lohi
        mid = lo + ((hi - lo + one) >> one)
        midv = jnp.full((16,), mid)

        def inner(j, acc, midv=midv):
            ks = candbuf[pl.ds(j * 16, 16)]
            return acc + plsc.all_reduce_population_count(ks >= midv)

        acc = lax.fori_loop(0, nvr, inner, jnp.zeros((16,), jnp.int32))
        ok = acc[0] >= k2
        return lax.select(ok, mid, lo), lax.select(ok, hi, mid - one)

    lo2, _ = lax.fori_loop(
        0, 16, l2, (t16 << 16, (t16 << 16) | jnp.uint32(0xFFFF))
    )
    return lo2


def _sc_search_kernel(keys_hbm, keys16_hbm, kmin_hbm, t_hbm, kbuf, kbuf16,
                      candbuf, kminbuf, tbuf):
    wid = lax.axis_index("s") * 2 + lax.axis_index("c")
    base = wid * _CLS_PER_W
    pltpu.sync_copy(kmin_hbm.at[pl.ds(base, _CLS_PER_W)], kminbuf)
    for grp in range(_CLS_PER_W // 16):
        kvec = kminbuf[pl.ds(grp * 16, 16)]
        tvec = jnp.zeros((16,), jnp.uint32)
        for chunk in range(16 // _CHUNK):
            cbase = base + grp * 16 + chunk * _CHUNK
            pltpu.sync_copy(keys_hbm.at[pl.ds(cbase, _CHUNK)], kbuf)
            pltpu.sync_copy(keys16_hbm.at[pl.ds(cbase, _CHUNK)], kbuf16)
            for cl in range(_CHUNK):
                lane = chunk * _CHUNK + cl
                t_cl = _sc_search_class(kbuf, kbuf16, candbuf, cl, kvec[lane])
                lane_ids = lax.iota(jnp.int32, 16)
                tvec = jnp.where(lane_ids == lane, t_cl, tvec)
        tbuf[pl.ds(grp * 16, 16)] = tvec
    pltpu.sync_copy(tbuf, t_hbm.at[pl.ds(base, _CLS_PER_W)])


_sc_search = functools.partial(
    pl.kernel,
    mesh=plsc.VectorSubcoreMesh(core_axis_name="c", subcore_axis_name="s"),
    out_type=jax.ShapeDtypeStruct((_COLS_P,), jnp.uint32),
    scratch_types=[
        pltpu.VMEM((_CHUNK, _ROWS), jnp.uint32),
        pltpu.VMEM((_CHUNK, _ROWS // 2), jnp.uint32),
        pltpu.VMEM((_ROWS + 32,), jnp.uint32),
        pltpu.VMEM((_CLS_PER_W,), jnp.int32),
        pltpu.VMEM((_CLS_PER_W,), jnp.uint32),
    ],
    compiler_params=pltpu.CompilerParams(needs_layout_passes=False),
)(_sc_search_kernel)


def _stage3_kernel(pred_ref, tgt_ref, posgt_ref, kmin_ref, t_ref, out_ref):
    i = pl.program_id(0)
    pred = pred_ref[...]
    tgt = tgt_ref[...]
    pos_gt = posgt_ref[0]
    kmin = kmin_ref[0]
    t = t_ref[0]

    _, majority, ukey = _keys_and_masks(pred, tgt)
    gt = ukey > t
    cnt_gt = jnp.sum(jnp.where(gt, 1, 0), axis=0, keepdims=True)
    loss = _bce(pred, tgt)
    sum_sel = jnp.sum(
        jnp.where(gt | (~majority), loss, 0.0), axis=0, keepdims=True
    )
    gval = _order_key_inv(t)
    pb = jnp.where(pos_gt > 0.5, -gval, gval)
    lossb = _bce(pb, pos_gt)
    tie = (kmin - cnt_gt).astype(jnp.float32)
    csum = jnp.where(kmin > 0, sum_sel + tie * lossb, 0.0)

    @pl.when(i == 0)
    def _():
        out_ref[0, 0] = 0.0

    out_ref[0, 0] += jnp.sum(csum)


def _stage3(pred_p, tgt_p, posgt3, kmin3, t3):
    return pl.pallas_call(
        _stage3_kernel,
        grid=(_COLS_P // _BLK_C,),
        in_specs=[
            pl.BlockSpec((_ROWS, _BLK_C), lambda i: (0, i)),
            pl.BlockSpec((_ROWS, _BLK_C), lambda i: (0, i)),
            pl.BlockSpec((1, 1, _BLK_C), lambda i: (i, 0, 0)),
            pl.BlockSpec((1, 1, _BLK_C), lambda i: (i, 0, 0)),
            pl.BlockSpec((1, 1, _BLK_C), lambda i: (i, 0, 0)),
        ],
        out_specs=pl.BlockSpec(memory_space=pltpu.SMEM),
        out_shape=jax.ShapeDtypeStruct((1, 1), jnp.float32),
        compiler_params=pltpu.CompilerParams(
            dimension_semantics=("arbitrary",)
        ),
    )(pred_p, tgt_p, posgt3, kmin3, t3)


@jax.jit
def kernel(pred, target):
    rows, cols = pred.shape
    pad = _COLS_P - cols
    pred_p = jnp.pad(pred, ((0, 0), (0, pad)))
    tgt_p = jnp.pad(target, ((0, 0), (0, pad)))

    keys_t, keys16_t, kmin3, posgt3 = _stage1(pred_p, tgt_p)
    keys16_pk = lax.bitcast_convert_type(
        keys16_t.reshape(_COLS_P, _ROWS // 2, 2), jnp.uint32
    )
    t_flat = _sc_search(keys_t, keys16_pk, kmin3.reshape(-1))
    t3 = t_flat.reshape(_COLS_P // _BLK_C, 1, _BLK_C)
    total = _stage3(pred_p, tgt_p, posgt3, kmin3, t3)
    return total[0, 0] / (rows * cols)


# speedup vs baseline: 1.4945x; 1.4945x over previous
import functools

import jax
import jax.numpy as jnp
import numpy as np
from jax import lax
from jax.experimental import pallas as pl
from jax.experimental.pallas import tpu as pltpu
from jax.experimental.pallas import tpu_sc as plsc

_ROWS = 4096
_COLS_P = 1024
_BLK_C = 128
_N_WORKERS = 32
_CLS_PER_W = _COLS_P // _N_WORKERS
_CHUNK = 8
_UNROLL = 16


def _bce(x, z):
    return jnp.maximum(x, 0.0) - x * z + jnp.log1p(jnp.exp(-jnp.abs(x)))


def _order_key(x):
    u = lax.bitcast_convert_type(x, jnp.uint32)
    neg = (u >> 31).astype(jnp.bool_)
    return jnp.where(neg, ~u, u | jnp.uint32(0x80000000))


def _order_key_inv(t):
    u = jnp.where(t >= jnp.uint32(0x80000000), t ^ jnp.uint32(0x80000000), ~t)
    return lax.bitcast_convert_type(u, jnp.float32)


def _keys_and_masks(pred, tgt):
    pos = jnp.sum(tgt, axis=0, keepdims=True)
    pos_gt = (pos >= (_ROWS / 2)).astype(pred.dtype)
    majority = tgt == pos_gt
    g = jnp.where(pos_gt > 0.5, -pred, pred)
    ukey = jnp.where(majority, _order_key(g), jnp.uint32(0))
    return pos_gt, majority, ukey


def _stage1_kernel(pred_ref, tgt_ref, keys_ref, kmin_ref, posgt_ref):
    pred = pred_ref[...]
    tgt = tgt_ref[...]
    pos_gt, majority, ukey = _keys_and_masks(pred, tgt)
    kmin = jnp.sum(jnp.where(majority, 0, 1), axis=0, keepdims=True)
    keys_ref[...] = ukey.T
    kmin_ref[...] = kmin.reshape(1, 1, _BLK_C)
    posgt_ref[...] = pos_gt.reshape(1, 1, _BLK_C)


def _stage1(pred_p, tgt_p):
    return pl.pallas_call(
        _stage1_kernel,
        grid=(_COLS_P // _BLK_C,),
        in_specs=[
            pl.BlockSpec((_ROWS, _BLK_C), lambda i: (0, i)),
            pl.BlockSpec((_ROWS, _BLK_C), lambda i: (0, i)),
        ],
        out_specs=[
            pl.BlockSpec((_BLK_C, _ROWS), lambda i: (i, 0)),
            pl.BlockSpec((1, 1, _BLK_C), lambda i: (i, 0, 0)),
            pl.BlockSpec((1, 1, _BLK_C), lambda i: (i, 0, 0)),
        ],
        out_shape=[
            jax.ShapeDtypeStruct((_COLS_P, _ROWS), jnp.uint32),
            jax.ShapeDtypeStruct((_COLS_P // _BLK_C, 1, _BLK_C), jnp.int32),
            jax.ShapeDtypeStruct((_COLS_P // _BLK_C, 1, _BLK_C), jnp.float32),
        ],
    )(pred_p, tgt_p)


def _sc_search_kernel(keys_hbm, kmin_hbm, t_hbm, kbuf, kminbuf, tbuf):
    wid = lax.axis_index("s") * 2 + lax.axis_index("c")
    base = wid * _CLS_PER_W
    pltpu.sync_copy(kmin_hbm.at[pl.ds(base, _CLS_PER_W)], kminbuf)
    for grp in range(_CLS_PER_W // 16):
        kvec = kminbuf[pl.ds(grp * 16, 16)]
        tvec = jnp.zeros((16,), jnp.uint32)
        for chunk in range(16 // _CHUNK):
            pltpu.sync_copy(
                keys_hbm.at[
                    pl.ds(base + grp * 16 + chunk * _CHUNK, _CHUNK)
                ],
                kbuf,
            )
            for cl in range(_CHUNK):
                lane = chunk * _CHUNK + cl
                kk = kvec[lane]

                def outer(_, lohi, cl=cl):
                    lo, hi = lohi
                    mid = lo + ((hi - lo + jnp.uint32(1)) >> jnp.uint32(1))
                    midv = jnp.full((16,), mid, jnp.uint32)

                    def inner(j, acc, cl=cl, midv=midv):
                        base = j * (16 * _UNROLL)
                        for u in range(_UNROLL):
                            ks = kbuf[cl, pl.ds(base + u * 16, 16)]
                            acc = acc + plsc.all_reduce_population_count(
                                ks >= midv
                            )
                        return acc

                    acc = lax.fori_loop(
                        0,
                        _ROWS // (16 * _UNROLL),
                        inner,
                        jnp.zeros((16,), jnp.int32),
                    )
                    cnt = acc[0]
                    ok = cnt >= kk
                    return (
                        lax.select(ok, mid, lo),
                        lax.select(ok, hi, mid - jnp.uint32(1)),
                    )

                lo, _ = lax.fori_loop(
                    0, 32, outer, (jnp.uint32(0), jnp.uint32(0xFF800000))
                )
                lane_ids = lax.iota(jnp.int32, 16)
                tvec = jnp.where(lane_ids == lane, lo, tvec)
        tbuf[pl.ds(grp * 16, 16)] = tvec
    pltpu.sync_copy(tbuf, t_hbm.at[pl.ds(base, _CLS_PER_W)])


_sc_search = functools.partial(
    pl.kernel,
    mesh=plsc.VectorSubcoreMesh(core_axis_name="c", subcore_axis_name="s"),
    out_type=jax.ShapeDtypeStruct((_COLS_P,), jnp.uint32),
    scratch_types=[
        pltpu.VMEM((_CHUNK, _ROWS), jnp.uint32),
        pltpu.VMEM((_CLS_PER_W,), jnp.int32),
        pltpu.VMEM((_CLS_PER_W,), jnp.uint32),
    ],
    compiler_params=pltpu.CompilerParams(needs_layout_passes=False),
)(_sc_search_kernel)


def _stage3_kernel(pred_ref, tgt_ref, posgt_ref, kmin_ref, t_ref, out_ref):
    i = pl.program_id(0)
    pred = pred_ref[...]
    tgt = tgt_ref[...]
    pos_gt = posgt_ref[0]
    kmin = kmin_ref[0]
    t = t_ref[0]

    _, majority, ukey = _keys_and_masks(pred, tgt)
    gt = ukey > t
    cnt_gt = jnp.sum(jnp.where(gt, 1, 0), axis=0, keepdims=True)
    loss = _bce(pred, tgt)
    sum_sel = jnp.sum(
        jnp.where(gt | (~majority), loss, 0.0), axis=0, keepdims=True
    )
    gval = _order_key_inv(t)
    pb = jnp.where(pos_gt > 0.5, -gval, gval)
    lossb = _bce(pb, pos_gt)
    tie = (kmin - cnt_gt).astype(jnp.float32)
    csum = jnp.where(kmin > 0, sum_sel + tie * lossb, 0.0)

    @pl.when(i == 0)
    def _():
        out_ref[0, 0] = 0.0

    out_ref[0, 0] += jnp.sum(csum)


def _stage3(pred_p, tgt_p, posgt3, kmin3, t3):
    return pl.pallas_call(
        _stage3_kernel,
        grid=(_COLS_P // _BLK_C,),
        in_specs=[
            pl.BlockSpec((_ROWS, _BLK_C), lambda i: (0, i)),
            pl.BlockSpec((_ROWS, _BLK_C), lambda i: (0, i)),
            pl.BlockSpec((1, 1, _BLK_C), lambda i: (i, 0, 0)),
            pl.BlockSpec((1, 1, _BLK_C), lambda i: (i, 0, 0)),
            pl.BlockSpec((1, 1, _BLK_C), lambda i: (i, 0, 0)),
        ],
        out_specs=pl.BlockSpec(memory_space=pltpu.SMEM),
        out_shape=jax.ShapeDtypeStruct((1, 1), jnp.float32),
        compiler_params=pltpu.CompilerParams(
            dimension_semantics=("arbitrary",)
        ),
    )(pred_p, tgt_p, posgt3, kmin3, t3)


@jax.jit
def kernel(pred, target):
    rows, cols = pred.shape
    pad = _COLS_P - cols
    pred_p = jnp.pad(pred, ((0, 0), (0, pad)))
    tgt_p = jnp.pad(target, ((0, 0), (0, pad)))

    keys_t, kmin3, posgt3 = _stage1(pred_p, tgt_p)
    t_flat = _sc_search(keys_t, kmin3.reshape(-1))
    t3 = t_flat.reshape(_COLS_P // _BLK_C, 1, _BLK_C)
    total = _stage3(pred_p, tgt_p, posgt3, kmin3, t3)
    return total[0, 0] / (rows * cols)


# speedup vs baseline: 1.6116x; 1.0784x over previous
import functools

import jax
import jax.numpy as jnp
from jax import lax
from jax.experimental import pallas as pl
from jax.experimental.pallas import tpu as pltpu
from jax.experimental.pallas import tpu_sc as plsc

_ROWS = 4096
_COLS_P = 1024
_BLK_C = 128
_N_WORKERS = 32
_CLS_PER_W = _COLS_P // _N_WORKERS
_CHUNK = 8
_UNROLL = 8


def _bce(x, z):
    return jnp.maximum(x, 0.0) - x * z + jnp.log1p(jnp.exp(-jnp.abs(x)))


def _order_key(x):
    u = lax.bitcast_convert_type(x, jnp.uint32)
    neg = (u >> 31).astype(jnp.bool_)
    return jnp.where(neg, ~u, u | jnp.uint32(0x80000000))


def _order_key_inv(t):
    u = jnp.where(t >= jnp.uint32(0x80000000), t ^ jnp.uint32(0x80000000), ~t)
    return lax.bitcast_convert_type(u, jnp.float32)


def _keys_and_masks(pred, tgt):
    pos = jnp.sum(tgt, axis=0, keepdims=True)
    pos_gt = (pos >= (_ROWS / 2)).astype(pred.dtype)
    majority = tgt == pos_gt
    g = jnp.where(pos_gt > 0.5, -pred, pred)
    ukey = jnp.where(majority, _order_key(g), jnp.uint32(0))
    return pos_gt, majority, ukey


def _valid_lanes(cols):
    col0 = pl.program_id(0) * _BLK_C
    lane = lax.broadcasted_iota(jnp.int32, (1, _BLK_C), 1)
    return (col0 + lane) < cols


def _stage1_kernel(pred_ref, tgt_ref, keys_ref, kmin_ref, posgt_ref, *,
                   cols):
    pred = pred_ref[...]
    tgt = tgt_ref[...]
    pos_gt, majority, ukey = _keys_and_masks(pred, tgt)
    kmin = jnp.sum(jnp.where(majority, 0, 1), axis=0, keepdims=True)
    kmin = jnp.where(_valid_lanes(cols), kmin, 0)
    keys_ref[...] = ukey.T
    kmin_ref[...] = kmin.reshape(1, 1, _BLK_C)
    posgt_ref[...] = pos_gt.reshape(1, 1, _BLK_C)


def _stage1(pred, tgt):
    return pl.pallas_call(
        functools.partial(_stage1_kernel, cols=pred.shape[1]),
        grid=(_COLS_P // _BLK_C,),
        in_specs=[
            pl.BlockSpec((_ROWS, _BLK_C), lambda i: (0, i)),
            pl.BlockSpec((_ROWS, _BLK_C), lambda i: (0, i)),
        ],
        out_specs=[
            pl.BlockSpec((_BLK_C, _ROWS), lambda i: (i, 0)),
            pl.BlockSpec((1, 1, _BLK_C), lambda i: (i, 0, 0)),
            pl.BlockSpec((1, 1, _BLK_C), lambda i: (i, 0, 0)),
        ],
        out_shape=[
            jax.ShapeDtypeStruct((_COLS_P, _ROWS), jnp.uint32),
            jax.ShapeDtypeStruct((_COLS_P // _BLK_C, 1, _BLK_C), jnp.int32),
            jax.ShapeDtypeStruct((_COLS_P // _BLK_C, 1, _BLK_C), jnp.float32),
        ],
    )(pred, tgt)


def _sc_search_kernel(keys_hbm, kmin_hbm, t_hbm, kbuf, kminbuf, tbuf):
    wid = lax.axis_index("s") * 2 + lax.axis_index("c")
    base = wid * _CLS_PER_W
    pltpu.sync_copy(kmin_hbm.at[pl.ds(base, _CLS_PER_W)], kminbuf)
    for grp in range(_CLS_PER_W // 16):
        kvec = kminbuf[pl.ds(grp * 16, 16)]
        tvec = jnp.zeros((16,), jnp.uint32)
        for chunk in range(16 // _CHUNK):
            pltpu.sync_copy(
                keys_hbm.at[
                    pl.ds(base + grp * 16 + chunk * _CHUNK, _CHUNK)
                ],
                kbuf,
            )
            for cl in range(_CHUNK):
                lane = chunk * _CHUNK + cl
                kk = kvec[lane]

                def outer(_, lohi, cl=cl):
                    lo, hi = lohi
                    mid = lo + ((hi - lo + jnp.uint32(1)) >> jnp.uint32(1))
                    midv = jnp.full((16,), mid, jnp.uint32)

                    def inner(j, acc, cl=cl, midv=midv):
                        base = j * (16 * _UNROLL)
                        for u in range(_UNROLL):
                            ks = kbuf[cl, pl.ds(base + u * 16, 16)]
                            acc = acc + plsc.all_reduce_population_count(
                                ks >= midv
                            )
                        return acc

                    acc = lax.fori_loop(
                        0,
                        _ROWS // (16 * _UNROLL),
                        inner,
                        jnp.zeros((16,), jnp.int32),
                    )
                    cnt = acc[0]
                    ok = cnt >= kk
                    return (
                        lax.select(ok, mid, lo),
                        lax.select(ok, hi, mid - jnp.uint32(1)),
                    )

                lo, _ = lax.fori_loop(
                    0, 32, outer, (jnp.uint32(0), jnp.uint32(0xFF800000))
                )
                lane_ids = lax.iota(jnp.int32, 16)
                tvec = jnp.where(lane_ids == lane, lo, tvec)
        tbuf[pl.ds(grp * 16, 16)] = tvec
    pltpu.sync_copy(tbuf, t_hbm.at[pl.ds(base, _CLS_PER_W)])


_sc_search = functools.partial(
    pl.kernel,
    mesh=plsc.VectorSubcoreMesh(core_axis_name="c", subcore_axis_name="s"),
    out_type=jax.ShapeDtypeStruct((_COLS_P,), jnp.uint32),
    scratch_types=[
        pltpu.VMEM((_CHUNK, _ROWS), jnp.uint32),
        pltpu.VMEM((_CLS_PER_W,), jnp.int32),
        pltpu.VMEM((_CLS_PER_W,), jnp.uint32),
    ],
    compiler_params=pltpu.CompilerParams(needs_layout_passes=False),
)(_sc_search_kernel)


def _stage3_kernel(pred_ref, tgt_ref, posgt_ref, kmin_ref, t_ref, out_ref, *,
                   cols):
    i = pl.program_id(0)
    pred = pred_ref[...]
    tgt = tgt_ref[...]
    pos_gt = posgt_ref[0]
    kmin = kmin_ref[0]
    t = t_ref[0]

    _, majority, ukey = _keys_and_masks(pred, tgt)
    gt = ukey > t
    cnt_gt = jnp.sum(jnp.where(gt, 1, 0), axis=0, keepdims=True)
    loss = _bce(pred, tgt)
    sum_sel = jnp.sum(
        jnp.where(gt | (~majority), loss, 0.0), axis=0, keepdims=True
    )
    gval = _order_key_inv(t)
    pb = jnp.where(pos_gt > 0.5, -gval, gval)
    lossb = _bce(pb, pos_gt)
    tie = (kmin - cnt_gt).astype(jnp.float32)
    keep = jnp.logical_and(kmin > 0, _valid_lanes(cols))
    csum = jnp.where(keep, sum_sel + tie * lossb, 0.0)

    @pl.when(i == 0)
    def _():
        out_ref[0, 0] = 0.0

    out_ref[0, 0] += jnp.sum(csum)


def _stage3(pred, tgt, posgt3, kmin3, t3):
    return pl.pallas_call(
        functools.partial(_stage3_kernel, cols=pred.shape[1]),
        grid=(_COLS_P // _BLK_C,),
        in_specs=[
            pl.BlockSpec((_ROWS, _BLK_C), lambda i: (0, i)),
            pl.BlockSpec((_ROWS, _BLK_C), lambda i: (0, i)),
            pl.BlockSpec((1, 1, _BLK_C), lambda i: (i, 0, 0)),
            pl.BlockSpec((1, 1, _BLK_C), lambda i: (i, 0, 0)),
            pl.BlockSpec((1, 1, _BLK_C), lambda i: (i, 0, 0)),
        ],
        out_specs=pl.BlockSpec(memory_space=pltpu.SMEM),
        out_shape=jax.ShapeDtypeStruct((1, 1), jnp.float32),
        compiler_params=pltpu.CompilerParams(
            dimension_semantics=("arbitrary",)
        ),
    )(pred, tgt, posgt3, kmin3, t3)


@jax.jit
def kernel(pred, target):
    rows, cols = pred.shape
    keys_t, kmin3, posgt3 = _stage1(pred, target)
    t_flat = _sc_search(keys_t, kmin3.reshape(-1))
    t3 = t_flat.reshape(_COLS_P // _BLK_C, 1, _BLK_C)
    total = _stage3(pred, target, posgt3, kmin3, t3)
    return total[0, 0] / (rows * cols)


# speedup vs baseline: 1.6484x; 1.0228x over previous
import functools

import jax
import jax.numpy as jnp
from jax import lax
from jax.experimental import pallas as pl
from jax.experimental.pallas import tpu as pltpu
from jax.experimental.pallas import tpu_sc as plsc

_ROWS = 4096
_COLS_P = 1024
_BLK_C = 128
_N_WORKERS = 32
_CLS_PER_W = _COLS_P // _N_WORKERS
_CHUNK = 8
_UNROLL = 8


def _bce(x, z):
    return jnp.maximum(x, 0.0) - x * z + jnp.log1p(jnp.exp(-jnp.abs(x)))


def _order_key(x):
    u = lax.bitcast_convert_type(x, jnp.uint32)
    neg = (u >> 31).astype(jnp.bool_)
    return jnp.where(neg, ~u, u | jnp.uint32(0x80000000))


def _order_key_inv(t):
    u = jnp.where(t >= jnp.uint32(0x80000000), t ^ jnp.uint32(0x80000000), ~t)
    return lax.bitcast_convert_type(u, jnp.float32)


def _keys_and_masks(pred, tgt):
    pos = jnp.sum(tgt, axis=0, keepdims=True)
    pos_gt = (pos >= (_ROWS / 2)).astype(pred.dtype)
    majority = tgt == pos_gt
    g = jnp.where(pos_gt > 0.5, -pred, pred)
    ukey = jnp.where(majority, _order_key(g), jnp.uint32(0))
    return pos_gt, majority, ukey


def _valid_lanes(cols):
    col0 = pl.program_id(0) * _BLK_C
    lane = lax.broadcasted_iota(jnp.int32, (1, _BLK_C), 1)
    return (col0 + lane) < cols


def _stage1_kernel(pred_ref, tgt_ref, keys_ref, kmin_ref, posgt_ref, *,
                   cols):
    pred = pred_ref[...]
    tgt = tgt_ref[...]
    pos_gt, majority, ukey = _keys_and_masks(pred, tgt)
    kmin = jnp.sum(jnp.where(majority, 0, 1), axis=0, keepdims=True)
    kmin = jnp.where(_valid_lanes(cols), kmin, 0)
    keys_ref[...] = ukey.T
    kmin_ref[...] = kmin.reshape(1, 1, _BLK_C)
    posgt_ref[...] = pos_gt.reshape(1, 1, _BLK_C)


def _stage1(pred, tgt):
    return pl.pallas_call(
        functools.partial(_stage1_kernel, cols=pred.shape[1]),
        grid=(_COLS_P // _BLK_C,),
        in_specs=[
            pl.BlockSpec((_ROWS, _BLK_C), lambda i: (0, i)),
            pl.BlockSpec((_ROWS, _BLK_C), lambda i: (0, i)),
        ],
        out_specs=[
            pl.BlockSpec((_BLK_C, _ROWS), lambda i: (i, 0)),
            pl.BlockSpec((1, 1, _BLK_C), lambda i: (i, 0, 0)),
            pl.BlockSpec((1, 1, _BLK_C), lambda i: (i, 0, 0)),
        ],
        out_shape=[
            jax.ShapeDtypeStruct((_COLS_P, _ROWS), jnp.uint32),
            jax.ShapeDtypeStruct((_COLS_P // _BLK_C, 1, _BLK_C), jnp.int32),
            jax.ShapeDtypeStruct((_COLS_P // _BLK_C, 1, _BLK_C), jnp.float32),
        ],
    )(pred, tgt)


def _sc_search_kernel(keys_hbm, kmin_hbm, t_hbm, kbuf0, kbuf1, kminbuf, tbuf,
                      sem0, sem1):
    wid = lax.axis_index("s") * 2 + lax.axis_index("c")
    base = wid * _CLS_PER_W
    pltpu.sync_copy(kmin_hbm.at[pl.ds(base, _CLS_PER_W)], kminbuf)
    bufs = [kbuf0, kbuf1]
    sems = [sem0, sem1]
    n_chunks = _CLS_PER_W // _CHUNK

    def _start(i):
        return pltpu.async_copy(
            keys_hbm.at[pl.ds(base + i * _CHUNK, _CHUNK)],
            bufs[i % 2],
            sems[i % 2],
        )

    pending = {0: _start(0)}
    for grp in range(_CLS_PER_W // 16):
        kvec = kminbuf[pl.ds(grp * 16, 16)]
        tvec = jnp.zeros((16,), jnp.uint32)
        for chunk in range(16 // _CHUNK):
            ci = grp * (16 // _CHUNK) + chunk
            pending.pop(ci).wait()
            if ci + 1 < n_chunks:
                pending[ci + 1] = _start(ci + 1)
            kbuf = bufs[ci % 2]
            for cl in range(_CHUNK):
                lane = chunk * _CHUNK + cl
                kk = kvec[lane]

                def outer(_, lohi, cl=cl):
                    lo, hi = lohi
                    mid = lo + ((hi - lo + jnp.uint32(1)) >> jnp.uint32(1))
                    midv = jnp.full((16,), mid, jnp.uint32)

                    def inner(j, acc, cl=cl, midv=midv):
                        base = j * (16 * _UNROLL)
                        for u in range(_UNROLL):
                            ks = kbuf[cl, pl.ds(base + u * 16, 16)]
                            acc = acc + plsc.all_reduce_population_count(
                                ks >= midv
                            )
                        return acc

                    acc = lax.fori_loop(
                        0,
                        _ROWS // (16 * _UNROLL),
                        inner,
                        jnp.zeros((16,), jnp.int32),
                    )
                    cnt = acc[0]
                    ok = cnt >= kk
                    return (
                        lax.select(ok, mid, lo),
                        lax.select(ok, hi, mid - jnp.uint32(1)),
                    )

                lo, _ = lax.fori_loop(
                    0, 32, outer, (jnp.uint32(0), jnp.uint32(0xFF800000))
                )
                lane_ids = lax.iota(jnp.int32, 16)
                tvec = jnp.where(lane_ids == lane, lo, tvec)
        tbuf[pl.ds(grp * 16, 16)] = tvec
    pltpu.sync_copy(tbuf, t_hbm.at[pl.ds(base, _CLS_PER_W)])


_sc_search = functools.partial(
    pl.kernel,
    mesh=plsc.VectorSubcoreMesh(core_axis_name="c", subcore_axis_name="s"),
    out_type=jax.ShapeDtypeStruct((_COLS_P,), jnp.uint32),
    scratch_types=[
        pltpu.VMEM((_CHUNK, _ROWS), jnp.uint32),
        pltpu.VMEM((_CHUNK, _ROWS), jnp.uint32),
        pltpu.VMEM((_CLS_PER_W,), jnp.int32),
        pltpu.VMEM((_CLS_PER_W,), jnp.uint32),
        pltpu.SemaphoreType.DMA,
        pltpu.SemaphoreType.DMA,
    ],
    compiler_params=pltpu.CompilerParams(needs_layout_passes=False),
)(_sc_search_kernel)


def _stage3_kernel(pred_ref, tgt_ref, posgt_ref, kmin_ref, t_ref, out_ref, *,
                   cols):
    i = pl.program_id(0)
    pred = pred_ref[...]
    tgt = tgt_ref[...]
    pos_gt = posgt_ref[0]
    kmin = kmin_ref[0]
    t = t_ref[0]

    _, majority, ukey = _keys_and_masks(pred, tgt)
    gt = ukey > t
    cnt_gt = jnp.sum(jnp.where(gt, 1, 0), axis=0, keepdims=True)
    loss = _bce(pred, tgt)
    sum_sel = jnp.sum(
        jnp.where(gt | (~majority), loss, 0.0), axis=0, keepdims=True
    )
    gval = _order_key_inv(t)
    pb = jnp.where(pos_gt > 0.5, -gval, gval)
    lossb = _bce(pb, pos_gt)
    tie = (kmin - cnt_gt).astype(jnp.float32)
    keep = jnp.logical_and(kmin > 0, _valid_lanes(cols))
    csum = jnp.where(keep, sum_sel + tie * lossb, 0.0)

    @pl.when(i == 0)
    def _():
        out_ref[0, 0] = 0.0

    out_ref[0, 0] += jnp.sum(csum)


def _stage3(pred, tgt, posgt3, kmin3, t3):
    return pl.pallas_call(
        functools.partial(_stage3_kernel, cols=pred.shape[1]),
        grid=(_COLS_P // _BLK_C,),
        in_specs=[
            pl.BlockSpec((_ROWS, _BLK_C), lambda i: (0, i)),
            pl.BlockSpec((_ROWS, _BLK_C), lambda i: (0, i)),
            pl.BlockSpec((1, 1, _BLK_C), lambda i: (i, 0, 0)),
            pl.BlockSpec((1, 1, _BLK_C), lambda i: (i, 0, 0)),
            pl.BlockSpec((1, 1, _BLK_C), lambda i: (i, 0, 0)),
        ],
        out_specs=pl.BlockSpec(memory_space=pltpu.SMEM),
        out_shape=jax.ShapeDtypeStruct((1, 1), jnp.float32),
        compiler_params=pltpu.CompilerParams(
            dimension_semantics=("arbitrary",)
        ),
    )(pred, tgt, posgt3, kmin3, t3)


@jax.jit
def kernel(pred, target):
    rows, cols = pred.shape
    keys_t, kmin3, posgt3 = _stage1(pred, target)
    t_flat = _sc_search(keys_t, kmin3.reshape(-1))
    t3 = t_flat.reshape(_COLS_P // _BLK_C, 1, _BLK_C)
    total = _stage3(pred, target, posgt3, kmin3, t3)
    return total[0, 0] / (rows * cols)
